# baseline (speedup 1.0000x reference)
import functools
import jax
import jax.numpy as jnp
from jax import lax
from jax.experimental import pallas as pl
from jax.experimental.pallas import tpu as pltpu
from jax.experimental.pallas import tpu_sc as plsc

BUCKET_SIZE = 64
N_HASHES = 4
TOKEN_SELF_ATTN_VALUE = -50000.0
NEG_MAX = -3.4028234663852886e38

CB = 16
NC = 2
NS = 16
NW = NC * NS


def _attn_body(qkv_ref, qkvp_ref, stq_ref, stk_ref, stkp_ref,
               sbq_ref, sbk_ref, sbkp_ref, o_ref):
    scale = 0.125
    qs, vs, kns = [], [], []
    for c in range(-1, CB):
        x = qkvp_ref[0, 0] if c == -1 else qkv_ref[0, c]
        k = x[:, :64]
        vv = x[:, 64:]
        nrm2 = jnp.sum(k * k, axis=1, keepdims=True)
        kn = k * lax.rsqrt(jnp.maximum(nrm2, 1e-24))
        qs.append(k)
        vs.append(vv)
        kns.append(kn)
    dn = (((1,), (1,)), ((), ()))
    for c in range(CB):
        q = qs[c + 1]
        do = lax.dot_general(q, kns[c + 1], dn,
                             preferred_element_type=jnp.float32) * scale
        dp = lax.dot_general(q, kns[c], dn,
                             preferred_element_type=jnp.float32) * scale
        st_q = stq_ref[0, c]
        sb_q = sbq_ref[0, c]
        st_ko = stk_ref[0, c]
        sb_ko = sbk_ref[0, c]
        if c == 0:
            st_kp = stkp_ref[0, 0]
            sb_kp = sbkp_ref[0, 0]
        else:
            st_kp = stk_ref[0, c - 1]
            sb_kp = sbk_ref[0, c - 1]
        do = jnp.where(sb_q != sb_ko, NEG_MAX,
                       jnp.where(st_q == st_ko, TOKEN_SELF_ATTN_VALUE, do))
        dp = jnp.where(sb_q != sb_kp, NEG_MAX,
                       jnp.where(st_q == st_kp, TOKEN_SELF_ATTN_VALUE, dp))
        rmax = jnp.maximum(jnp.max(do, axis=1, keepdims=True),
                           jnp.max(dp, axis=1, keepdims=True))
        eo = jnp.exp(do - rmax)
        ep = jnp.exp(dp - rmax)
        ssum = (jnp.sum(eo, axis=1, keepdims=True)
                + jnp.sum(ep, axis=1, keepdims=True))
        o = (lax.dot_general(eo, vs[c + 1], (((1,), (0,)), ((), ())),
                             preferred_element_type=jnp.float32)
             + lax.dot_general(ep, vs[c], (((1,), (0,)), ((), ())),
                               preferred_element_type=jnp.float32))
        o_ref[0, c, :, :64] = o
        o_ref[0, c, :, 64:65] = rmax
        o_ref[0, c, :, 65:66] = ssum


def _combine_body(x_ref, out_ref):
    xs = [x_ref[0, h] for h in range(N_HASHES)]
    rs = [x[:, 64:65] for x in xs]
    ss = [x[:, 65:66] for x in xs]
    m = rs[0]
    for h in range(1, N_HASHES):
        m = jnp.maximum(m, rs[h])
    ws = [jnp.exp(r - m) for r in rs]
    den = ws[0] * ss[0]
    for h in range(1, N_HASHES):
        den = den + ws[h] * ss[h]
    acc = ws[0] * xs[0][:, :64]
    for h in range(1, N_HASHES):
        acc = acc + ws[h] * xs[h][:, :64]
    out_ref[0] = acc / den


def _sc_gather_body(tab_hbm, idx_hbm, out_hbm, idx_v, rows_v, sem):
    wid = lax.axis_index("s") * NC + lax.axis_index("c")
    n_idx_rows = idx_hbm.shape[0]
    rows_per_w = n_idx_rows // NW
    idx_base = wid * rows_per_w
    pltpu.sync_copy(idx_hbm.at[pl.ds(idx_base, rows_per_w)], idx_v)

    GROUP = 4

    def group_body(g, _):
        handles = []
        for j in range(GROUP):
            idx_row = idx_v.at[g * GROUP + j]
            handles.append(pltpu.async_copy(
                tab_hbm.at[idx_row], rows_v.at[pl.ds(j * 128, 128)], sem))
        for h in handles:
            h.wait()
        out_base = (idx_base + g * GROUP) * 128
        pltpu.sync_copy(rows_v, out_hbm.at[pl.ds(out_base, GROUP * 128)])
        return ()

    lax.fori_loop(0, rows_per_w // GROUP, group_body, ())


def _sc_gather(tab, idx_flat):
    total, dd = idx_flat.shape[0], tab.shape[1]
    idx2d = idx_flat.reshape(total // 128, 128)
    mesh = plsc.VectorSubcoreMesh(core_axis_name="c", subcore_axis_name="s")
    fn = pl.kernel(
        _sc_gather_body,
        mesh=mesh,
        out_type=jax.ShapeDtypeStruct((total, dd), jnp.float32),
        scratch_types=[
            pltpu.VMEM((total // 128 // NW, 128), jnp.int32),
            pltpu.VMEM((512, dd), jnp.float32),
            pltpu.SemaphoreType.DMA,
        ],
    )
    return fn(tab, idx2d)


def kernel(qk, v, random_rotations):
    b, s, d = qk.shape
    n_buckets = s // BUCKET_SIZE
    nh = N_HASHES
    nchunk = nh * n_buckets
    cw = (nh * s) // nchunk

    rotated = jnp.einsum('btf,fhi->bhti', qk, random_rotations[0])
    rotated = jnp.concatenate([rotated, -rotated], axis=-1)
    buckets = jnp.argmax(rotated, axis=-1).astype(jnp.int32)
    offsets = (jnp.arange(nh, dtype=jnp.int32) * n_buckets).reshape(1, nh, 1)
    buckets = (buckets + offsets).reshape(b, nh * s)

    ticker = jnp.broadcast_to(jnp.arange(nh * s, dtype=jnp.int32)[None, :],
                              buckets.shape)
    buckets_and_t = s * buckets + (ticker % s)
    sidx = jnp.argsort(buckets_and_t, axis=-1).astype(jnp.int32)
    sticker = jnp.take_along_axis(ticker, sidx, axis=-1)
    undo_sort = jnp.argsort(sticker, axis=-1).astype(jnp.int32)
    sbuckets = jnp.take_along_axis(buckets, sidx, axis=-1)
    st = sticker % s

    st_glob = st + (jnp.arange(b, dtype=jnp.int32) * s)[:, None]
    qkv_tab = jnp.concatenate([qk, v], axis=-1).reshape(b * s, 2 * d)
    sqkv_f = _sc_gather(qkv_tab, st_glob.reshape(-1))
    sqkv = sqkv_f.reshape(b, nchunk, cw, 2 * d)

    st_c = st.reshape(b, nchunk, cw)
    sb_c = sbuckets.reshape(b, nchunk, cw)

    st_col = st_c[:, :, :, None]
    sb_col = sb_c[:, :, :, None]
    st_rowm = st_c[:, :, None, :]
    sb_rowm = sb_c[:, :, None, :]

    nb = nchunk // CB
    grid = (b, nb)
    attn = pl.pallas_call(
        _attn_body,
        grid=grid,
        in_specs=[
            pl.BlockSpec((1, CB, cw, 2 * d), lambda bi, ci: (bi, ci, 0, 0)),
            pl.BlockSpec((1, 1, cw, 2 * d),
                         lambda bi, ci: (bi, (ci * CB - 1) % nchunk, 0, 0)),
            pl.BlockSpec((1, CB, cw, 1), lambda bi, ci: (bi, ci, 0, 0)),
            pl.BlockSpec((1, CB, 1, cw), lambda bi, ci: (bi, ci, 0, 0)),
            pl.BlockSpec((1, 1, 1, cw),
                         lambda bi, ci: (bi, (ci * CB - 1) % nchunk, 0, 0)),
            pl.BlockSpec((1, CB, cw, 1), lambda bi, ci: (bi, ci, 0, 0)),
            pl.BlockSpec((1, CB, 1, cw), lambda bi, ci: (bi, ci, 0, 0)),
            pl.BlockSpec((1, 1, 1, cw),
                         lambda bi, ci: (bi, (ci * CB - 1) % nchunk, 0, 0)),
        ],
        out_specs=[
            pl.BlockSpec((1, CB, cw, 2 * d), lambda bi, ci: (bi, ci, 0, 0)),
        ],
        out_shape=[
            jax.ShapeDtypeStruct((b, nchunk, cw, 2 * d), jnp.float32),
        ],
    )
    (so2,) = attn(sqkv, sqkv, st_col, st_rowm, st_rowm,
                  sb_col, sb_rowm, sb_rowm)

    undo_glob = undo_sort + (jnp.arange(b, dtype=jnp.int32) * (nh * s))[:, None]
    o_u2 = _sc_gather(so2.reshape(b * nh * s, 2 * d), undo_glob.reshape(-1))
    o_u2 = o_u2.reshape(b, nh, s, 2 * d)

    SB = 512
    comb = pl.pallas_call(
        _combine_body,
        grid=(b, s // SB),
        in_specs=[
            pl.BlockSpec((1, nh, SB, 2 * d), lambda bi, si: (bi, 0, si, 0)),
        ],
        out_specs=pl.BlockSpec((1, SB, d), lambda bi, si: (bi, si, 0)),
        out_shape=jax.ShapeDtypeStruct((b, s, d), jnp.float32),
    )
    return comb(o_u2)


# speedup vs baseline: 1.0442x; 1.0442x over previous
import functools
import jax
import jax.numpy as jnp
from jax import lax
from jax.experimental import pallas as pl
from jax.experimental.pallas import tpu as pltpu
from jax.experimental.pallas import tpu_sc as plsc

BUCKET_SIZE = 64
N_HASHES = 4
TOKEN_SELF_ATTN_VALUE = -50000.0
NEG_MAX = -3.4028234663852886e38

CB = 16
NC = 2
NS = 16
NW = NC * NS


def _attn_body(qkv_ref, qkvp_ref, stq_ref, stk_ref, stkp_ref,
               sbq_ref, sbk_ref, sbkp_ref, o_ref):
    scale = 0.125
    qs, vs, kns = [], [], []
    for c in range(-1, CB):
        x = qkvp_ref[0, 0] if c == -1 else qkv_ref[0, c]
        k = x[:, :64]
        vv = x[:, 64:]
        nrm2 = jnp.sum(k * k, axis=1, keepdims=True)
        kn = k * lax.rsqrt(jnp.maximum(nrm2, 1e-24))
        qs.append(k)
        vs.append(vv)
        kns.append(kn)
    dn = (((1,), (1,)), ((), ()))
    for c in range(CB):
        q = qs[c + 1]
        kcat = jnp.concatenate([kns[c + 1], kns[c]], axis=0)
        vcat = jnp.concatenate([vs[c + 1], vs[c]], axis=0)
        dots = lax.dot_general(q, kcat, dn,
                               preferred_element_type=jnp.float32) * scale
        st_q = stq_ref[0, c]
        sb_q = sbq_ref[0, c]
        if c == 0:
            st_kp = stkp_ref[0, 0]
            sb_kp = sbkp_ref[0, 0]
        else:
            st_kp = stk_ref[0, c - 1]
            sb_kp = sbk_ref[0, c - 1]
        st_k = jnp.concatenate([stk_ref[0, c], st_kp], axis=1)
        sb_k = jnp.concatenate([sbk_ref[0, c], sb_kp], axis=1)
        dots = jnp.where(sb_q != sb_k, NEG_MAX,
                         jnp.where(st_q == st_k, TOKEN_SELF_ATTN_VALUE, dots))
        rmax = jnp.max(dots, axis=1, keepdims=True)
        e = jnp.exp(dots - rmax)
        ssum = jnp.sum(e, axis=1, keepdims=True)
        o = lax.dot_general(e, vcat, (((1,), (0,)), ((), ())),
                            preferred_element_type=jnp.float32)
        o_ref[0, c, :, :64] = o
        o_ref[0, c, :, 64:65] = rmax
        o_ref[0, c, :, 65:66] = ssum


def _combine_body(x_ref, out_ref):
    xs = [x_ref[0, h] for h in range(N_HASHES)]
    rs = [x[:, 64:65] for x in xs]
    ss = [x[:, 65:66] for x in xs]
    m = rs[0]
    for h in range(1, N_HASHES):
        m = jnp.maximum(m, rs[h])
    ws = [jnp.exp(r - m) for r in rs]
    den = ws[0] * ss[0]
    for h in range(1, N_HASHES):
        den = den + ws[h] * ss[h]
    acc = ws[0] * xs[0][:, :64]
    for h in range(1, N_HASHES):
        acc = acc + ws[h] * xs[h][:, :64]
    out_ref[0] = acc / den


def _sc_gather_body(tab_hbm, idx_hbm, out_hbm, idx_v, rows_v, sem):
    wid = lax.axis_index("s") * NC + lax.axis_index("c")
    n_idx_rows = idx_hbm.shape[0]
    rows_per_w = n_idx_rows // NW
    idx_base = wid * rows_per_w
    pltpu.sync_copy(idx_hbm.at[pl.ds(idx_base, rows_per_w)], idx_v)

    GROUP = 4

    def group_body(g, _):
        handles = []
        for j in range(GROUP):
            idx_row = idx_v.at[g * GROUP + j]
            handles.append(pltpu.async_copy(
                tab_hbm.at[idx_row], rows_v.at[pl.ds(j * 128, 128)], sem))
        for h in handles:
            h.wait()
        out_base = (idx_base + g * GROUP) * 128
        pltpu.sync_copy(rows_v, out_hbm.at[pl.ds(out_base, GROUP * 128)])
        return ()

    lax.fori_loop(0, rows_per_w // GROUP, group_body, ())


def _sc_gather(tab, idx_flat):
    total, dd = idx_flat.shape[0], tab.shape[1]
    idx2d = idx_flat.reshape(total // 128, 128)
    mesh = plsc.VectorSubcoreMesh(core_axis_name="c", subcore_axis_name="s")
    fn = pl.kernel(
        _sc_gather_body,
        mesh=mesh,
        out_type=jax.ShapeDtypeStruct((total, dd), jnp.float32),
        scratch_types=[
            pltpu.VMEM((total // 128 // NW, 128), jnp.int32),
            pltpu.VMEM((512, dd), jnp.float32),
            pltpu.SemaphoreType.DMA,
        ],
    )
    return fn(tab, idx2d)


def kernel(qk, v, random_rotations):
    b, s, d = qk.shape
    n_buckets = s // BUCKET_SIZE
    nh = N_HASHES
    nchunk = nh * n_buckets
    cw = (nh * s) // nchunk

    rotated = jnp.einsum('btf,fhi->bhti', qk, random_rotations[0])
    rotated = jnp.concatenate([rotated, -rotated], axis=-1)
    buckets = jnp.argmax(rotated, axis=-1).astype(jnp.int32)
    offsets = (jnp.arange(nh, dtype=jnp.int32) * n_buckets).reshape(1, nh, 1)
    buckets = (buckets + offsets).reshape(b, nh * s)

    ticker = jnp.broadcast_to(jnp.arange(nh * s, dtype=jnp.int32)[None, :],
                              buckets.shape)
    buckets_and_t = s * buckets + (ticker % s)
    sidx = jnp.argsort(buckets_and_t, axis=-1).astype(jnp.int32)
    sticker = jnp.take_along_axis(ticker, sidx, axis=-1)
    undo_sort = jnp.argsort(sticker, axis=-1).astype(jnp.int32)
    sbuckets = jnp.take_along_axis(buckets, sidx, axis=-1)
    st = sticker % s

    st_glob = st + (jnp.arange(b, dtype=jnp.int32) * s)[:, None]
    qkv_tab = jnp.concatenate([qk, v], axis=-1).reshape(b * s, 2 * d)
    sqkv_f = _sc_gather(qkv_tab, st_glob.reshape(-1))
    sqkv = sqkv_f.reshape(b, nchunk, cw, 2 * d)

    st_c = st.reshape(b, nchunk, cw)
    sb_c = sbuckets.reshape(b, nchunk, cw)

    st_col = st_c[:, :, :, None]
    sb_col = sb_c[:, :, :, None]
    st_rowm = st_c[:, :, None, :]
    sb_rowm = sb_c[:, :, None, :]

    nb = nchunk // CB
    grid = (b, nb)
    attn = pl.pallas_call(
        _attn_body,
        grid=grid,
        in_specs=[
            pl.BlockSpec((1, CB, cw, 2 * d), lambda bi, ci: (bi, ci, 0, 0)),
            pl.BlockSpec((1, 1, cw, 2 * d),
                         lambda bi, ci: (bi, (ci * CB - 1) % nchunk, 0, 0)),
            pl.BlockSpec((1, CB, cw, 1), lambda bi, ci: (bi, ci, 0, 0)),
            pl.BlockSpec((1, CB, 1, cw), lambda bi, ci: (bi, ci, 0, 0)),
            pl.BlockSpec((1, 1, 1, cw),
                         lambda bi, ci: (bi, (ci * CB - 1) % nchunk, 0, 0)),
            pl.BlockSpec((1, CB, cw, 1), lambda bi, ci: (bi, ci, 0, 0)),
            pl.BlockSpec((1, CB, 1, cw), lambda bi, ci: (bi, ci, 0, 0)),
            pl.BlockSpec((1, 1, 1, cw),
                         lambda bi, ci: (bi, (ci * CB - 1) % nchunk, 0, 0)),
        ],
        out_specs=[
            pl.BlockSpec((1, CB, cw, 2 * d), lambda bi, ci: (bi, ci, 0, 0)),
        ],
        out_shape=[
            jax.ShapeDtypeStruct((b, nchunk, cw, 2 * d), jnp.float32),
        ],
    )
    (so2,) = attn(sqkv, sqkv, st_col, st_rowm, st_rowm,
                  sb_col, sb_rowm, sb_rowm)

    undo_glob = undo_sort + (jnp.arange(b, dtype=jnp.int32) * (nh * s))[:, None]
    o_u2 = _sc_gather(so2.reshape(b * nh * s, 2 * d), undo_glob.reshape(-1))
    o_u2 = o_u2.reshape(b, nh, s, 2 * d)

    SB = 512
    comb = pl.pallas_call(
        _combine_body,
        grid=(b, s // SB),
        in_specs=[
            pl.BlockSpec((1, nh, SB, 2 * d), lambda bi, si: (bi, 0, si, 0)),
        ],
        out_specs=pl.BlockSpec((1, SB, d), lambda bi, si: (bi, si, 0)),
        out_shape=jax.ShapeDtypeStruct((b, s, d), jnp.float32),
    )
    return comb(o_u2)


# speedup vs baseline: 1.0645x; 1.0194x over previous
import functools
import jax
import jax.numpy as jnp
from jax import lax
from jax.experimental import pallas as pl
from jax.experimental.pallas import tpu as pltpu
from jax.experimental.pallas import tpu_sc as plsc

BUCKET_SIZE = 64
N_HASHES = 4
TOKEN_SELF_ATTN_VALUE = -50000.0
NEG_MAX = -3.4028234663852886e38

CB = 32
NC = 2
NS = 16
NW = NC * NS


def _attn_body(qkv_ref, qkvp_ref, stq_ref, stk_ref, stkp_ref,
               sbq_ref, sbk_ref, sbkp_ref, o_ref):
    scale = 0.125
    qs, vs, kns = [], [], []
    for c in range(-1, CB):
        x = qkvp_ref[0, 0] if c == -1 else qkv_ref[0, c]
        k = x[:, :64]
        vv = x[:, 64:]
        nrm2 = jnp.sum(k * k, axis=1, keepdims=True)
        kn = k * lax.rsqrt(jnp.maximum(nrm2, 1e-24))
        qs.append(k)
        vs.append(vv)
        kns.append(kn)
    dn = (((1,), (1,)), ((), ()))
    for c in range(CB):
        q = qs[c + 1]
        kcat = jnp.concatenate([kns[c + 1], kns[c]], axis=0)
        vcat = jnp.concatenate([vs[c + 1], vs[c]], axis=0)
        dots = lax.dot_general(q, kcat, dn,
                               preferred_element_type=jnp.float32) * scale
        st_q = stq_ref[0, c]
        sb_q = sbq_ref[0, c]
        if c == 0:
            st_kp = stkp_ref[0, 0]
            sb_kp = sbkp_ref[0, 0]
        else:
            st_kp = stk_ref[0, c - 1]
            sb_kp = sbk_ref[0, c - 1]
        st_k = jnp.concatenate([stk_ref[0, c], st_kp], axis=1)
        sb_k = jnp.concatenate([sbk_ref[0, c], sb_kp], axis=1)
        dots = jnp.where(sb_q != sb_k, NEG_MAX,
                         jnp.where(st_q == st_k, TOKEN_SELF_ATTN_VALUE, dots))
        rmax = jnp.max(dots, axis=1, keepdims=True)
        e = jnp.exp(dots - rmax)
        ssum = jnp.sum(e, axis=1, keepdims=True)
        o = lax.dot_general(e, vcat, (((1,), (0,)), ((), ())),
                            preferred_element_type=jnp.float32)
        o_ref[0, c, :, :64] = o
        o_ref[0, c, :, 64:65] = rmax
        o_ref[0, c, :, 65:66] = ssum


def _combine_body(x_ref, out_ref):
    xs = [x_ref[0, h] for h in range(N_HASHES)]
    rs = [x[:, 64:65] for x in xs]
    ss = [x[:, 65:66] for x in xs]
    m = rs[0]
    for h in range(1, N_HASHES):
        m = jnp.maximum(m, rs[h])
    ws = [jnp.exp(r - m) for r in rs]
    den = ws[0] * ss[0]
    for h in range(1, N_HASHES):
        den = den + ws[h] * ss[h]
    acc = ws[0] * xs[0][:, :64]
    for h in range(1, N_HASHES):
        acc = acc + ws[h] * xs[h][:, :64]
    out_ref[0] = acc / den


def _sc_gather_body(tab_hbm, idx_hbm, out_hbm, idx_v, rows_v, sem):
    wid = lax.axis_index("s") * NC + lax.axis_index("c")
    n_idx_rows = idx_hbm.shape[0]
    rows_per_w = n_idx_rows // NW
    idx_base = wid * rows_per_w
    pltpu.sync_copy(idx_hbm.at[pl.ds(idx_base, rows_per_w)], idx_v)

    GROUP = 4

    def group_body(g, _):
        handles = []
        for j in range(GROUP):
            idx_row = idx_v.at[g * GROUP + j]
            handles.append(pltpu.async_copy(
                tab_hbm.at[idx_row], rows_v.at[pl.ds(j * 128, 128)], sem))
        for h in handles:
            h.wait()
        out_base = (idx_base + g * GROUP) * 128
        pltpu.sync_copy(rows_v, out_hbm.at[pl.ds(out_base, GROUP * 128)])
        return ()

    lax.fori_loop(0, rows_per_w // GROUP, group_body, ())


def _sc_gather(tab, idx_flat):
    total, dd = idx_flat.shape[0], tab.shape[1]
    idx2d = idx_flat.reshape(total // 128, 128)
    mesh = plsc.VectorSubcoreMesh(core_axis_name="c", subcore_axis_name="s")
    fn = pl.kernel(
        _sc_gather_body,
        mesh=mesh,
        out_type=jax.ShapeDtypeStruct((total, dd), jnp.float32),
        scratch_types=[
            pltpu.VMEM((total // 128 // NW, 128), jnp.int32),
            pltpu.VMEM((512, dd), jnp.float32),
            pltpu.SemaphoreType.DMA,
        ],
    )
    return fn(tab, idx2d)


def kernel(qk, v, random_rotations):
    b, s, d = qk.shape
    n_buckets = s // BUCKET_SIZE
    nh = N_HASHES
    nchunk = nh * n_buckets
    cw = (nh * s) // nchunk

    rotated = jnp.einsum('btf,fhi->bhti', qk, random_rotations[0])
    rotated = jnp.concatenate([rotated, -rotated], axis=-1)
    buckets = jnp.argmax(rotated, axis=-1).astype(jnp.int32)
    offsets = (jnp.arange(nh, dtype=jnp.int32) * n_buckets).reshape(1, nh, 1)
    buckets = (buckets + offsets).reshape(b, nh * s)

    ticker = jnp.broadcast_to(jnp.arange(nh * s, dtype=jnp.int32)[None, :],
                              buckets.shape)
    buckets_and_t = s * buckets + (ticker % s)
    sidx = jnp.argsort(buckets_and_t, axis=-1).astype(jnp.int32)
    sticker = jnp.take_along_axis(ticker, sidx, axis=-1)
    undo_sort = jnp.argsort(sticker, axis=-1).astype(jnp.int32)
    sbuckets = jnp.take_along_axis(buckets, sidx, axis=-1)
    st = sticker % s

    st_glob = st + (jnp.arange(b, dtype=jnp.int32) * s)[:, None]
    qkv_tab = jnp.concatenate([qk, v], axis=-1).reshape(b * s, 2 * d)
    sqkv_f = _sc_gather(qkv_tab, st_glob.reshape(-1))
    sqkv = sqkv_f.reshape(b, nchunk, cw, 2 * d)

    st_c = st.reshape(b, nchunk, cw)
    sb_c = sbuckets.reshape(b, nchunk, cw)

    st_col = st_c[:, :, :, None]
    sb_col = sb_c[:, :, :, None]
    st_rowm = st_c[:, :, None, :]
    sb_rowm = sb_c[:, :, None, :]

    nb = nchunk // CB
    grid = (b, nb)
    attn = pl.pallas_call(
        _attn_body,
        grid=grid,
        in_specs=[
            pl.BlockSpec((1, CB, cw, 2 * d), lambda bi, ci: (bi, ci, 0, 0)),
            pl.BlockSpec((1, 1, cw, 2 * d),
                         lambda bi, ci: (bi, (ci * CB - 1) % nchunk, 0, 0)),
            pl.BlockSpec((1, CB, cw, 1), lambda bi, ci: (bi, ci, 0, 0)),
            pl.BlockSpec((1, CB, 1, cw), lambda bi, ci: (bi, ci, 0, 0)),
            pl.BlockSpec((1, 1, 1, cw),
                         lambda bi, ci: (bi, (ci * CB - 1) % nchunk, 0, 0)),
            pl.BlockSpec((1, CB, cw, 1), lambda bi, ci: (bi, ci, 0, 0)),
            pl.BlockSpec((1, CB, 1, cw), lambda bi, ci: (bi, ci, 0, 0)),
            pl.BlockSpec((1, 1, 1, cw),
                         lambda bi, ci: (bi, (ci * CB - 1) % nchunk, 0, 0)),
        ],
        out_specs=[
            pl.BlockSpec((1, CB, cw, 2 * d), lambda bi, ci: (bi, ci, 0, 0)),
        ],
        out_shape=[
            jax.ShapeDtypeStruct((b, nchunk, cw, 2 * d), jnp.float32),
        ],
    )
    (so2,) = attn(sqkv, sqkv, st_col, st_rowm, st_rowm,
                  sb_col, sb_rowm, sb_rowm)

    undo_glob = undo_sort + (jnp.arange(b, dtype=jnp.int32) * (nh * s))[:, None]
    o_u2 = _sc_gather(so2.reshape(b * nh * s, 2 * d), undo_glob.reshape(-1))
    o_u2 = o_u2.reshape(b, nh, s, 2 * d)

    SB = 512
    comb = pl.pallas_call(
        _combine_body,
        grid=(b, s // SB),
        in_specs=[
            pl.BlockSpec((1, nh, SB, 2 * d), lambda bi, si: (bi, 0, si, 0)),
        ],
        out_specs=pl.BlockSpec((1, SB, d), lambda bi, si: (bi, si, 0)),
        out_shape=jax.ShapeDtypeStruct((b, s, d), jnp.float32),
    )
    return comb(o_u2)


# speedup vs baseline: 1.1288x; 1.0605x over previous
import functools
import jax
import jax.numpy as jnp
from jax import lax
from jax.experimental import pallas as pl
from jax.experimental.pallas import tpu as pltpu
from jax.experimental.pallas import tpu_sc as plsc

BUCKET_SIZE = 64
N_HASHES = 4
TOKEN_SELF_ATTN_VALUE = -50000.0
NEG_MAX = -3.4028234663852886e38

CB = 32
NC = 2
NS = 16
NW = NC * NS


def _attn_body(qkv_ref, qkvp_ref, stq_ref, stk_ref, stkp_ref,
               sbq_ref, sbk_ref, sbkp_ref, o_ref):
    scale = 0.125
    qs, vs, kns = [], [], []
    for c in range(-1, CB):
        x = qkvp_ref[0, 0] if c == -1 else qkv_ref[0, c]
        k = x[:, :64]
        vv = x[:, 64:]
        nrm2 = jnp.sum(k * k, axis=1, keepdims=True)
        kn = k * lax.rsqrt(jnp.maximum(nrm2, 1e-24))
        qs.append(k)
        vs.append(vv)
        kns.append(kn)
    dn = (((1,), (1,)), ((), ()))
    for c in range(CB):
        q = qs[c + 1]
        kcat = jnp.concatenate([kns[c + 1], kns[c]], axis=0)
        vcat = jnp.concatenate([vs[c + 1], vs[c]], axis=0)
        dots = lax.dot_general(q, kcat, dn,
                               preferred_element_type=jnp.float32) * scale
        st_q = stq_ref[0, c]
        sb_q = sbq_ref[0, c]
        if c == 0:
            st_kp = stkp_ref[0, 0]
            sb_kp = sbkp_ref[0, 0]
        else:
            st_kp = stk_ref[0, c - 1]
            sb_kp = sbk_ref[0, c - 1]
        st_k = jnp.concatenate([stk_ref[0, c], st_kp], axis=1)
        sb_k = jnp.concatenate([sbk_ref[0, c], sb_kp], axis=1)
        dots = jnp.where(sb_q != sb_k, NEG_MAX,
                         jnp.where(st_q == st_k, TOKEN_SELF_ATTN_VALUE, dots))
        rmax = jnp.max(dots, axis=1, keepdims=True)
        e = jnp.exp(dots - rmax)
        ssum = jnp.sum(e, axis=1, keepdims=True)
        o = lax.dot_general(e, vcat, (((1,), (0,)), ((), ())),
                            preferred_element_type=jnp.float32)
        o_ref[0, c, :, :64] = o
        o_ref[0, c, :, 64:65] = rmax
        o_ref[0, c, :, 65:66] = ssum


def _combine_body(x_ref, out_ref):
    xs = [x_ref[0, h] for h in range(N_HASHES)]
    rs = [x[:, 64:65] for x in xs]
    ss = [x[:, 65:66] for x in xs]
    m = rs[0]
    for h in range(1, N_HASHES):
        m = jnp.maximum(m, rs[h])
    ws = [jnp.exp(r - m) for r in rs]
    den = ws[0] * ss[0]
    for h in range(1, N_HASHES):
        den = den + ws[h] * ss[h]
    acc = ws[0] * xs[0][:, :64]
    for h in range(1, N_HASHES):
        acc = acc + ws[h] * xs[h][:, :64]
    out_ref[0] = acc / den


def _sc_gather_body(tab_hbm, idx_hbm, out_hbm, idx_v, rows_v, sem):
    wid = lax.axis_index("s") * NC + lax.axis_index("c")
    n_idx_rows = idx_hbm.shape[0]
    rows_per_w = n_idx_rows // NW
    idx_base = wid * rows_per_w
    pltpu.sync_copy(idx_hbm.at[pl.ds(idx_base, rows_per_w)], idx_v)

    GROUP = 4

    def group_body(g, _):
        handles = []
        for j in range(GROUP):
            idx_row = idx_v.at[g * GROUP + j]
            handles.append(pltpu.async_copy(
                tab_hbm.at[idx_row], rows_v.at[pl.ds(j * 128, 128)], sem))
        for h in handles:
            h.wait()
        out_base = (idx_base + g * GROUP) * 128
        pltpu.sync_copy(rows_v, out_hbm.at[pl.ds(out_base, GROUP * 128)])
        return ()

    lax.fori_loop(0, rows_per_w // GROUP, group_body, ())


def _sc_scatter_body(src_hbm, idx_hbm, out_hbm, idx_v, rows_v, sem):
    wid = lax.axis_index("s") * NC + lax.axis_index("c")
    n_idx_rows = idx_hbm.shape[0]
    rows_per_w = n_idx_rows // NW
    idx_base = wid * rows_per_w
    pltpu.sync_copy(idx_hbm.at[pl.ds(idx_base, rows_per_w)], idx_v)

    GROUP = 4

    def group_body(g, _):
        src_base = (idx_base + g * GROUP) * 128
        pltpu.sync_copy(src_hbm.at[pl.ds(src_base, GROUP * 128)], rows_v)
        handles = []
        for j in range(GROUP):
            idx_row = idx_v.at[g * GROUP + j]
            handles.append(pltpu.async_copy(
                rows_v.at[pl.ds(j * 128, 128)], out_hbm.at[idx_row], sem))
        for h in handles:
            h.wait()
        return ()

    lax.fori_loop(0, rows_per_w // GROUP, group_body, ())


def _sc_scatter(src, idx_flat):
    total, dd = idx_flat.shape[0], src.shape[1]
    idx2d = idx_flat.reshape(total // 128, 128)
    mesh = plsc.VectorSubcoreMesh(core_axis_name="c", subcore_axis_name="s")
    fn = pl.kernel(
        _sc_scatter_body,
        mesh=mesh,
        out_type=jax.ShapeDtypeStruct((total, dd), jnp.float32),
        scratch_types=[
            pltpu.VMEM((total // 128 // NW, 128), jnp.int32),
            pltpu.VMEM((512, dd), jnp.float32),
            pltpu.SemaphoreType.DMA,
        ],
    )
    return fn(src, idx2d)


def _sc_gather(tab, idx_flat):
    total, dd = idx_flat.shape[0], tab.shape[1]
    idx2d = idx_flat.reshape(total // 128, 128)
    mesh = plsc.VectorSubcoreMesh(core_axis_name="c", subcore_axis_name="s")
    fn = pl.kernel(
        _sc_gather_body,
        mesh=mesh,
        out_type=jax.ShapeDtypeStruct((total, dd), jnp.float32),
        scratch_types=[
            pltpu.VMEM((total // 128 // NW, 128), jnp.int32),
            pltpu.VMEM((512, dd), jnp.float32),
            pltpu.SemaphoreType.DMA,
        ],
    )
    return fn(tab, idx2d)


def kernel(qk, v, random_rotations):
    b, s, d = qk.shape
    n_buckets = s // BUCKET_SIZE
    nh = N_HASHES
    nchunk = nh * n_buckets
    cw = (nh * s) // nchunk

    rotated = jnp.einsum('btf,fhi->bhti', qk, random_rotations[0])
    rotated = jnp.concatenate([rotated, -rotated], axis=-1)
    buckets = jnp.argmax(rotated, axis=-1).astype(jnp.int32)
    offsets = (jnp.arange(nh, dtype=jnp.int32) * n_buckets).reshape(1, nh, 1)
    buckets = (buckets + offsets).reshape(b, nh * s)

    ticker = jnp.arange(nh * s, dtype=jnp.int32)[None, :]
    buckets_and_t = s * buckets + (ticker % s)
    sidx = jnp.argsort(buckets_and_t, axis=-1).astype(jnp.int32)
    sbuckets = jnp.take_along_axis(buckets, sidx, axis=-1)
    st = sidx % s

    st_glob = st + (jnp.arange(b, dtype=jnp.int32) * s)[:, None]
    qkv_tab = jnp.concatenate([qk, v], axis=-1).reshape(b * s, 2 * d)
    sqkv_f = _sc_gather(qkv_tab, st_glob.reshape(-1))
    sqkv = sqkv_f.reshape(b, nchunk, cw, 2 * d)

    st_c = st.reshape(b, nchunk, cw)
    sb_c = sbuckets.reshape(b, nchunk, cw)

    st_col = st_c[:, :, :, None]
    sb_col = sb_c[:, :, :, None]
    st_rowm = st_c[:, :, None, :]
    sb_rowm = sb_c[:, :, None, :]

    nb = nchunk // CB
    grid = (b, nb)
    attn = pl.pallas_call(
        _attn_body,
        grid=grid,
        in_specs=[
            pl.BlockSpec((1, CB, cw, 2 * d), lambda bi, ci: (bi, ci, 0, 0)),
            pl.BlockSpec((1, 1, cw, 2 * d),
                         lambda bi, ci: (bi, (ci * CB - 1) % nchunk, 0, 0)),
            pl.BlockSpec((1, CB, cw, 1), lambda bi, ci: (bi, ci, 0, 0)),
            pl.BlockSpec((1, CB, 1, cw), lambda bi, ci: (bi, ci, 0, 0)),
            pl.BlockSpec((1, 1, 1, cw),
                         lambda bi, ci: (bi, (ci * CB - 1) % nchunk, 0, 0)),
            pl.BlockSpec((1, CB, cw, 1), lambda bi, ci: (bi, ci, 0, 0)),
            pl.BlockSpec((1, CB, 1, cw), lambda bi, ci: (bi, ci, 0, 0)),
            pl.BlockSpec((1, 1, 1, cw),
                         lambda bi, ci: (bi, (ci * CB - 1) % nchunk, 0, 0)),
        ],
        out_specs=[
            pl.BlockSpec((1, CB, cw, 2 * d), lambda bi, ci: (bi, ci, 0, 0)),
        ],
        out_shape=[
            jax.ShapeDtypeStruct((b, nchunk, cw, 2 * d), jnp.float32),
        ],
    )
    (so2,) = attn(sqkv, sqkv, st_col, st_rowm, st_rowm,
                  sb_col, sb_rowm, sb_rowm)

    total = b * nh * s
    dest = (jnp.arange(total, dtype=jnp.int32) // s) * s + st.reshape(-1)
    o_u2 = _sc_scatter(so2.reshape(total, 2 * d), dest)
    o_u2 = o_u2.reshape(b, nh, s, 2 * d)

    SB = 512
    comb = pl.pallas_call(
        _combine_body,
        grid=(b, s // SB),
        in_specs=[
            pl.BlockSpec((1, nh, SB, 2 * d), lambda bi, si: (bi, 0, si, 0)),
        ],
        out_specs=pl.BlockSpec((1, SB, d), lambda bi, si: (bi, si, 0)),
        out_shape=jax.ShapeDtypeStruct((b, s, d), jnp.float32),
    )
    return comb(o_u2)
